# Initial kernel scaffold; baseline (speedup 1.0000x reference)
import jax
import jax.numpy as jnp
from jax.experimental import pallas as pl


def kernel(f, pm, convFS_w, convFS_b, resFS_conv1_w, resFS_conv1_b, resFS_conv2_w, resFS_conv2_b, resMM_conv1_w, resMM_conv1_b, resMM_conv2_w, resMM_conv2_b):
    raise NotImplementedError("write your pallas kernel here")


# speedup vs baseline: 2.5987x; 2.5987x over previous
import functools

import jax
import jax.numpy as jnp
from jax import lax
from jax.experimental import pallas as pl
from jax.experimental.pallas import tpu as pltpu

_VMEM_LIMIT = 56 * 1024 * 1024
_PAD = 64


def _interp_mat(out_size, in_size):
    if out_size == 1:
        src = jnp.zeros((out_size,), jnp.float32)
    else:
        src = jnp.arange(out_size, dtype=jnp.float32) * (
            (in_size - 1) / (out_size - 1))
    i0 = jnp.clip(jnp.floor(src), 0, in_size - 1).astype(jnp.int32)
    i1 = jnp.clip(i0 + 1, 0, in_size - 1)
    frac = src - i0.astype(jnp.float32)
    return (jax.nn.one_hot(i0, in_size, dtype=jnp.float32) * (1.0 - frac)[:, None]
            + jax.nn.one_hot(i1, in_size, dtype=jnp.float32) * frac[:, None])


def _refine_kernel(f_ref, pm_ref, wfs_ref, bfs_ref,
                   w11_ref, b11_ref, w12_ref, b12_ref,
                   w21_ref, b21_ref, w22_ref, b22_ref,
                   kup_ref, o_ref, xpad_f, xpad_c, *, H, W):
    HW = H * W
    col = lax.broadcasted_iota(jnp.int32, (1, HW), 1) % W
    mask_l = col != 0
    mask_r = col != (W - 1)

    def conv3x3(xpad, v_bf16, wt_ref, b_ref):
        cin = v_bf16.shape[0]
        xpad[:, 0:_PAD] = jnp.zeros((cin, _PAD), jnp.bfloat16)
        xpad[:, _PAD + HW:] = jnp.zeros((cin, _PAD), jnp.bfloat16)
        xpad[:, _PAD:_PAD + HW] = v_bf16
        acc = None
        for t, (dy, dx) in enumerate((dy, dx) for dy in (-1, 0, 1)
                                     for dx in (-1, 0, 1)):
            s = dy * W + dx
            p = xpad[:, _PAD + s:_PAD + s + HW]
            if dx == -1:
                p = jnp.where(mask_l, p, jnp.bfloat16(0))
            elif dx == 1:
                p = jnp.where(mask_r, p, jnp.bfloat16(0))
            d = lax.dot_general(wt_ref[t], p, (((1,), (0,)), ((), ())),
                                preferred_element_type=jnp.float32)
            acc = d if acc is None else acc + d
        return acc + b_ref[...]

    x = f_ref[0].astype(jnp.bfloat16)
    h1 = conv3x3(xpad_f, x, wfs_ref, bfs_ref)

    r = conv3x3(xpad_c, jnp.maximum(h1, 0.0).astype(jnp.bfloat16),
                w11_ref, b11_ref)
    r = conv3x3(xpad_c, jnp.maximum(r, 0.0).astype(jnp.bfloat16),
                w12_ref, b12_ref)
    s = h1 + r

    up = lax.dot_general(pm_ref[0], kup_ref[...], (((1,), (0,)), ((), ())),
                         preferred_element_type=jnp.float32)
    m = s + up

    r2 = conv3x3(xpad_c, jnp.maximum(m, 0.0).astype(jnp.bfloat16),
                 w21_ref, b21_ref)
    r2 = conv3x3(xpad_c, jnp.maximum(r2, 0.0).astype(jnp.bfloat16),
                 w22_ref, b22_ref)
    o_ref[0] = m + r2


def kernel(f, pm, convFS_w, convFS_b,
           resFS_conv1_w, resFS_conv1_b, resFS_conv2_w, resFS_conv2_b,
           resMM_conv1_w, resMM_conv1_b, resMM_conv2_w, resMM_conv2_b):
    N, Cin, H, W = f.shape
    _, C, h, w = pm.shape
    HW, hw = H * W, h * w

    f_flat = f.reshape(N, Cin, HW)
    pm_flat = pm.reshape(N, C, hw)

    def prep_w(wc):
        ci, co = wc.shape[2], wc.shape[3]
        return jnp.transpose(wc.reshape(9, ci, co), (0, 2, 1)).astype(jnp.bfloat16)

    def prep_b(bc):
        return bc.reshape(-1, 1)

    ah = _interp_mat(H, h)
    aw = _interp_mat(W, w)
    kup = jnp.einsum('Yy,Xx->yxYX', ah, aw).reshape(hw, HW)

    whole = lambda shp: pl.BlockSpec(shp, lambda n: (0,) * len(shp))

    out = pl.pallas_call(
        functools.partial(_refine_kernel, H=H, W=W),
        out_shape=jax.ShapeDtypeStruct((N, C, HW), jnp.float32),
        grid=(N,),
        in_specs=[
            pl.BlockSpec((1, Cin, HW), lambda n: (n, 0, 0)),
            pl.BlockSpec((1, C, hw), lambda n: (n, 0, 0)),
            whole((9, C, Cin)), whole((C, 1)),
            whole((9, C, C)), whole((C, 1)),
            whole((9, C, C)), whole((C, 1)),
            whole((9, C, C)), whole((C, 1)),
            whole((9, C, C)), whole((C, 1)),
            whole((hw, HW)),
        ],
        out_specs=pl.BlockSpec((1, C, HW), lambda n: (n, 0, 0)),
        scratch_shapes=[
            pltpu.VMEM((Cin, HW + 2 * _PAD), jnp.bfloat16),
            pltpu.VMEM((C, HW + 2 * _PAD), jnp.bfloat16),
        ],
        compiler_params=pltpu.CompilerParams(
            dimension_semantics=("parallel",),
            vmem_limit_bytes=_VMEM_LIMIT),
    )(f_flat, pm_flat,
      prep_w(convFS_w), prep_b(convFS_b),
      prep_w(resFS_conv1_w), prep_b(resFS_conv1_b),
      prep_w(resFS_conv2_w), prep_b(resFS_conv2_b),
      prep_w(resMM_conv1_w), prep_b(resMM_conv1_b),
      prep_w(resMM_conv2_w), prep_b(resMM_conv2_b),
      kup)
    return out.reshape(N, C, H, W)
